# baseline (speedup 1.0000x reference)
import functools

import jax
import jax.numpy as jnp
from jax import lax
from jax.experimental import pallas as pl
from jax.experimental.pallas import tpu as pltpu
from jax.experimental.pallas import tpu_sc as plsc

N = 10000
NP = 10240
E = 320000
D_IN = 128
D_HID = 24
DP = 32
NC, NS = 2, 16
NW = NC * NS
CHUNK = 128
KJ = 80
EP = NW * KJ * CHUNK
RPT = NP // NS


def _mesh():
    return plsc.VectorSubcoreMesh(core_axis_name="c", subcore_axis_name="s",
                                  num_cores=NC, num_subcores=NS)


def _rsqrt16(d):
    d = jnp.maximum(d, 1.0)
    i = plsc.bitcast(d, jnp.int32)
    i = 0x5F3759DF - lax.shift_right_logical(i, 1)
    y = plsc.bitcast(i, jnp.float32)
    for _ in range(3):
        y = y * (1.5 - 0.5 * d * y * y)
    return y


def _conv_loop(hw_sp, accum, src_v, dst_v, rows0, rows1, sem0, sem1):
    pltpu.async_copy(hw_sp.at[src_v.at[0]], rows0, sem0)

    @pl.loop(0, KJ // 2)
    def _(p):
        j = 2 * p
        pltpu.make_async_copy(hw_sp.at[src_v.at[j]], rows0, sem0).wait()
        pltpu.async_copy(hw_sp.at[src_v.at[j + 1]], rows1, sem1)
        pltpu.sync_copy(rows0, accum.at[dst_v.at[j]], add=True)

        @pl.when(j + 2 < KJ)
        def _():
            pltpu.async_copy(hw_sp.at[src_v.at[j + 2]], rows0, sem0)

        pltpu.make_async_copy(hw_sp.at[src_v.at[j + 1]], rows1, sem1).wait()
        pltpu.sync_copy(rows1, accum.at[dst_v.at[j + 1]], add=True)


@functools.cache
def _make_fused_kernel():
    @functools.partial(
        pl.kernel,
        out_type=(jax.ShapeDtypeStruct((NC, 2, NP), jnp.float32),
                  jax.ShapeDtypeStruct((NC, NP, DP), jnp.float32)),
        mesh=_mesh(),
        scratch_types=[
            pltpu.VMEM((2, KJ, CHUNK), jnp.int32),
            pltpu.VMEM((KJ, CHUNK), jnp.int32),
            pltpu.VMEM((CHUNK,), jnp.float32),
            pltpu.VMEM((RPT, DP), jnp.float32),
            pltpu.VMEM((RPT,), jnp.float32),
            pltpu.VMEM((CHUNK, DP), jnp.float32),
            pltpu.VMEM((CHUNK, DP), jnp.float32),
            pltpu.VMEM_SHARED((NP,), jnp.float32),
            pltpu.VMEM_SHARED((NP,), jnp.float32),
            pltpu.VMEM_SHARED((NP, DP), jnp.float32),
            pltpu.VMEM_SHARED((NP, DP), jnp.float32),
            pltpu.SemaphoreType.DMA,
            pltpu.SemaphoreType.DMA,
        ],
        compiler_params=pltpu.CompilerParams(use_tc_tiling_on_sc=False,
                                             needs_layout_passes=False),
    )
    def _fused(xw, srcw, dstw, zrows, zeros_hbm, ones_hbm, deg_out, agg_out,
               src_v, dst_v, ones_v, buf, s_v, rows0, rows1, dego, degi,
               hw_sp, accum, sem0, sem1):
        c = lax.axis_index("c")
        s = lax.axis_index("s")
        w = c * NS + s
        base = s * RPT

        pltpu.sync_copy(zeros_hbm.at[pl.ds(base, RPT)],
                        dego.at[pl.ds(base, RPT)])
        pltpu.sync_copy(zeros_hbm.at[pl.ds(base, RPT)],
                        degi.at[pl.ds(base, RPT)])
        pltpu.sync_copy(zrows.at[pl.ds(base, RPT)],
                        accum.at[pl.ds(base, RPT)])
        pltpu.sync_copy(ones_hbm, ones_v)
        pltpu.sync_copy(srcw.at[s], src_v.at[0])
        pltpu.sync_copy(srcw.at[NS + s], src_v.at[1])
        pltpu.sync_copy(dstw.at[w], dst_v)
        pltpu.sync_copy(xw.at[pl.ds(base, RPT)], buf)
        plsc.subcore_barrier()

        @pl.loop(0, KJ)
        def _(j):
            pltpu.sync_copy(ones_v, dego.at[src_v.at[0, j]], add=True)
            pltpu.sync_copy(ones_v, dego.at[src_v.at[1, j]], add=True)
            pltpu.sync_copy(ones_v, degi.at[dst_v.at[j]], add=True)

        plsc.subcore_barrier()

        pltpu.sync_copy(dego.at[pl.ds(base, RPT)],
                        deg_out.at[c, 0, pl.ds(base, RPT)])
        pltpu.sync_copy(degi.at[pl.ds(base, RPT)],
                        deg_out.at[c, 1, pl.ds(base, RPT)])
        pltpu.sync_copy(dego.at[pl.ds(base, RPT)], s_v)

        @pl.loop(0, RPT // 16)
        def _(k):
            s_v[pl.ds(k * 16, 16)] = _rsqrt16(s_v[pl.ds(k * 16, 16)])

        @pl.loop(0, RPT)
        def _(r):
            sv = plsc.load_gather(s_v, [jnp.full((16,), r, jnp.int32)])
            buf[r, pl.ds(0, 16)] = buf[r, pl.ds(0, 16)] * sv
            buf[r, pl.ds(16, 16)] = buf[r, pl.ds(16, 16)] * sv

        pltpu.sync_copy(buf, hw_sp.at[pl.ds(base, RPT)])
        plsc.subcore_barrier()

        _conv_loop(hw_sp, accum, src_v.at[c], dst_v, rows0, rows1, sem0, sem1)
        plsc.subcore_barrier()
        pltpu.sync_copy(accum.at[pl.ds(base, RPT)],
                        agg_out.at[c, pl.ds(base, RPT)])

    return _fused


@functools.cache
def _make_conv_kernel():
    @functools.partial(
        pl.kernel,
        out_type=jax.ShapeDtypeStruct((NC, NP, DP), jnp.float32),
        mesh=_mesh(),
        scratch_types=[
            pltpu.VMEM((KJ, CHUNK), jnp.int32),
            pltpu.VMEM((KJ, CHUNK), jnp.int32),
            pltpu.VMEM((CHUNK, DP), jnp.float32),
            pltpu.VMEM((CHUNK, DP), jnp.float32),
            pltpu.VMEM_SHARED((NP, DP), jnp.float32),
            pltpu.VMEM_SHARED((NP, DP), jnp.float32),
            pltpu.SemaphoreType.DMA,
            pltpu.SemaphoreType.DMA,
        ],
        compiler_params=pltpu.CompilerParams(use_tc_tiling_on_sc=False),
    )
    def _conv_kernel(hw, srcw, dstw, zrows, out, src_v, dst_v, rows0, rows1,
                     accum, hw_sp, sem0, sem1):
        c = lax.axis_index("c")
        s = lax.axis_index("s")
        w = c * NS + s
        base = s * RPT
        pltpu.sync_copy(zrows.at[pl.ds(base, RPT)],
                        accum.at[pl.ds(base, RPT)])
        pltpu.sync_copy(hw.at[pl.ds(base, RPT)], hw_sp.at[pl.ds(base, RPT)])
        pltpu.sync_copy(srcw.at[w], src_v)
        pltpu.sync_copy(dstw.at[w], dst_v)
        plsc.subcore_barrier()
        _conv_loop(hw_sp, accum, src_v, dst_v, rows0, rows1, sem0, sem1)
        plsc.subcore_barrier()
        pltpu.sync_copy(accum.at[pl.ds(base, RPT)],
                        out.at[c, pl.ds(base, RPT)])

    return _conv_kernel


RB = 1024


def _tc_xw(xp, W1p):
    def body(x_ref, w_ref, o_ref):
        o_ref[...] = jnp.dot(x_ref[...], w_ref[...],
                             preferred_element_type=jnp.float32)

    return pl.pallas_call(
        body,
        grid=(NP // RB,),
        in_specs=[
            pl.BlockSpec((RB, D_IN), lambda i: (i, 0)),
            pl.BlockSpec((D_IN, DP), lambda i: (0, 0)),
        ],
        out_specs=pl.BlockSpec((RB, DP), lambda i: (i, 0)),
        out_shape=jax.ShapeDtypeStruct((NP, DP), jnp.float32),
    )(xp, W1p)


def _scales(d):
    so = lax.rsqrt(jnp.maximum(d[:, 0:1], 1.0))
    si = lax.rsqrt(jnp.maximum(d[:, 1:2] + d[:, 3:4], 1.0))
    return so, si


def _tc_mid(a0, a1, deg4, b1p, W2p):
    def body(a0_ref, a1_ref, d_ref, b_ref, w_ref, o_ref):
        so, si = _scales(d_ref[...])
        a = a0_ref[...] + a1_ref[...]
        h = jnp.maximum(a * si + b_ref[...], 0.0)
        o_ref[...] = jnp.dot(h, w_ref[...],
                             preferred_element_type=jnp.float32) * so

    return pl.pallas_call(
        body,
        grid=(NP // RB,),
        in_specs=[
            pl.BlockSpec((RB, DP), lambda i: (i, 0)),
            pl.BlockSpec((RB, DP), lambda i: (i, 0)),
            pl.BlockSpec((RB, 4), lambda i: (i, 0)),
            pl.BlockSpec((1, DP), lambda i: (0, 0)),
            pl.BlockSpec((DP, DP), lambda i: (0, 0)),
        ],
        out_specs=pl.BlockSpec((RB, DP), lambda i: (i, 0)),
        out_shape=jax.ShapeDtypeStruct((NP, DP), jnp.float32),
    )(a0, a1, deg4, b1p, W2p)


def _tc_post(a0, a1, deg4, b2p):
    def body(a0_ref, a1_ref, d_ref, b_ref, o_ref):
        _, si = _scales(d_ref[...])
        a = a0_ref[...] + a1_ref[...]
        o_ref[...] = jnp.maximum(a * si + b_ref[...], 0.0)

    return pl.pallas_call(
        body,
        grid=(NP // RB,),
        in_specs=[
            pl.BlockSpec((RB, DP), lambda i: (i, 0)),
            pl.BlockSpec((RB, DP), lambda i: (i, 0)),
            pl.BlockSpec((RB, 4), lambda i: (i, 0)),
            pl.BlockSpec((1, DP), lambda i: (0, 0)),
        ],
        out_specs=pl.BlockSpec((RB, DP), lambda i: (i, 0)),
        out_shape=jax.ShapeDtypeStruct((NP, DP), jnp.float32),
    )(a0, a1, deg4, b2p)


def _tc_head(xrp, WdP, bdP):
    def body(x_ref, w_ref, b_ref, o_ref):
        o_ref[...] = jnp.dot(x_ref[...], w_ref[...],
                             preferred_element_type=jnp.float32) + b_ref[...]

    return pl.pallas_call(
        body,
        in_specs=[
            pl.BlockSpec((2560, 4 * D_HID), lambda: (0, 0)),
            pl.BlockSpec((4 * D_HID, 8), lambda: (0, 0)),
            pl.BlockSpec((1, 8), lambda: (0, 0)),
        ],
        out_specs=pl.BlockSpec((2560, 8), lambda: (0, 0)),
        out_shape=jax.ShapeDtypeStruct((2560, 8), jnp.float32),
    )(xrp, WdP, bdP)


def kernel(x, edge_index, W1, b1, W2, b2, Wd, bd):
    f32 = jnp.float32
    src = edge_index[0].astype(jnp.int32)
    dst = edge_index[1].astype(jnp.int32)
    pad = EP - E
    src_t = jnp.concatenate([src, jnp.full((pad,), N, jnp.int32)]
                            ).reshape(NW, KJ, CHUNK)
    dst_t = jnp.concatenate([dst, jnp.full((pad,), N, jnp.int32)]
                            ).reshape(NW, KJ, CHUNK)
    ones128 = jnp.ones((CHUNK,), f32)
    zerosN = jnp.zeros((NP,), f32)
    zrows = jnp.zeros((NP, DP), f32)

    xp = jnp.pad(x, ((0, NP - N), (0, 0)))
    W1p = jnp.pad(W1, ((0, 0), (0, DP - D_HID)))
    W2p = jnp.pad(W2, ((0, DP - D_HID), (0, DP - D_HID)))
    b1p = jnp.pad(b1, (0, DP - D_HID)).reshape(1, DP)
    b2p = jnp.pad(b2, (0, DP - D_HID)).reshape(1, DP)

    xw = _tc_xw(xp, W1p)
    deg, agg1 = _make_fused_kernel()(xw, src_t, dst_t, zrows, zerosN, ones128)
    deg4 = deg.transpose(2, 0, 1).reshape(NP, 2 * NC)
    hw2 = _tc_mid(agg1[0], agg1[1], deg4, b1p, W2p)
    agg2 = _make_conv_kernel()(hw2, src_t, dst_t, zrows)
    h2 = _tc_post(agg2[0], agg2[1], deg4, b2p)

    xr = h2[:N, :D_HID].reshape(N // 4, 4 * D_HID)
    xrp = jnp.pad(xr, ((0, 2560 - N // 4), (0, 0)))
    WdP = jnp.pad(Wd, ((0, 0), (0, 7)))
    bdP = jnp.pad(bd, (0, 7)).reshape(1, 8)
    out = _tc_head(xrp, WdP, bdP)
    return out[:N // 4, :1]


# speedup vs baseline: 1.1242x; 1.1242x over previous
import functools

import jax
import jax.numpy as jnp
from jax import lax
from jax.experimental import pallas as pl
from jax.experimental.pallas import tpu as pltpu
from jax.experimental.pallas import tpu_sc as plsc

N = 10000
NP = 10240
E = 320000
D_IN = 128
D_HID = 24
NC, NS = 2, 16
NW = NC * NS
CHUNK = 128
KJ = 80
EP = NW * KJ * CHUNK
RPT = NP // NS


def _mesh():
    return plsc.VectorSubcoreMesh(core_axis_name="c", subcore_axis_name="s",
                                  num_cores=NC, num_subcores=NS)


@functools.cache
def _make_degree_kernel():
    @functools.partial(
        pl.kernel,
        out_type=jax.ShapeDtypeStruct((NC, 2, NP), jnp.float32),
        mesh=_mesh(),
        scratch_types=[
            pltpu.VMEM((KJ, CHUNK), jnp.int32),
            pltpu.VMEM((KJ, CHUNK), jnp.int32),
            pltpu.VMEM((CHUNK,), jnp.float32),
            pltpu.VMEM_SHARED((NP,), jnp.float32),
            pltpu.VMEM_SHARED((NP,), jnp.float32),
        ],
        compiler_params=pltpu.CompilerParams(use_tc_tiling_on_sc=False),
    )
    def _degree_kernel(srcw, dstw, ones_hbm, zeros_hbm, out, src_v, dst_v,
                       ones_v, dego, degi):
        c = lax.axis_index("c")
        s = lax.axis_index("s")
        w = c * NS + s
        base = s * RPT
        pltpu.sync_copy(zeros_hbm.at[pl.ds(base, RPT)],
                        dego.at[pl.ds(base, RPT)])
        pltpu.sync_copy(zeros_hbm.at[pl.ds(base, RPT)],
                        degi.at[pl.ds(base, RPT)])
        pltpu.sync_copy(ones_hbm, ones_v)
        pltpu.sync_copy(srcw.at[w], src_v)
        pltpu.sync_copy(dstw.at[w], dst_v)
        plsc.subcore_barrier()

        @pl.loop(0, KJ)
        def _(j):
            pltpu.sync_copy(ones_v, dego.at[src_v.at[j]], add=True)
            pltpu.sync_copy(ones_v, degi.at[dst_v.at[j]], add=True)

        plsc.subcore_barrier()
        pltpu.sync_copy(dego.at[pl.ds(base, RPT)],
                        out.at[c, 0, pl.ds(base, RPT)])
        pltpu.sync_copy(degi.at[pl.ds(base, RPT)],
                        out.at[c, 1, pl.ds(base, RPT)])

    return _degree_kernel


@functools.cache
def _make_conv_kernel():
    @functools.partial(
        pl.kernel,
        out_type=jax.ShapeDtypeStruct((NC, NP, D_HID), jnp.float32),
        mesh=_mesh(),
        scratch_types=[
            pltpu.VMEM((KJ, CHUNK), jnp.int32),
            pltpu.VMEM((KJ, CHUNK), jnp.int32),
            pltpu.VMEM((CHUNK, D_HID), jnp.float32),
            pltpu.VMEM((CHUNK, D_HID), jnp.float32),
            pltpu.VMEM_SHARED((NP, D_HID), jnp.float32),
            pltpu.VMEM_SHARED((NP, D_HID), jnp.float32),
            pltpu.SemaphoreType.DMA,
            pltpu.SemaphoreType.DMA,
        ],
        compiler_params=pltpu.CompilerParams(use_tc_tiling_on_sc=False),
    )
    def _conv_kernel(hw, srcw, dstw, zrows, out, src_v, dst_v, rows0, rows1,
                     accum, hw_sp, sem0, sem1):
        c = lax.axis_index("c")
        s = lax.axis_index("s")
        w = c * NS + s
        base = s * RPT
        pltpu.sync_copy(zrows.at[pl.ds(base, RPT)],
                        accum.at[pl.ds(base, RPT)])
        pltpu.sync_copy(hw.at[pl.ds(base, RPT)], hw_sp.at[pl.ds(base, RPT)])
        pltpu.sync_copy(srcw.at[w], src_v)
        pltpu.sync_copy(dstw.at[w], dst_v)
        plsc.subcore_barrier()

        pltpu.async_copy(hw_sp.at[src_v.at[0]], rows0, sem0)

        @pl.loop(0, KJ // 2)
        def _(p):
            j = 2 * p
            pltpu.make_async_copy(hw_sp.at[src_v.at[j]], rows0, sem0).wait()
            pltpu.async_copy(hw_sp.at[src_v.at[j + 1]], rows1, sem1)
            pltpu.sync_copy(rows0, accum.at[dst_v.at[j]], add=True)

            @pl.when(j + 2 < KJ)
            def _():
                pltpu.async_copy(hw_sp.at[src_v.at[j + 2]], rows0, sem0)

            pltpu.make_async_copy(hw_sp.at[src_v.at[j + 1]], rows1,
                                  sem1).wait()
            pltpu.sync_copy(rows1, accum.at[dst_v.at[j + 1]], add=True)

        plsc.subcore_barrier()
        pltpu.sync_copy(accum.at[pl.ds(base, RPT)],
                        out.at[c, pl.ds(base, RPT)])

    return _conv_kernel


RB = 2560


def _tc_layer1(xp, W1, deg4):
    def body(x_ref, w_ref, d_ref, hw_ref, s_ref):
        d = d_ref[...]
        so = lax.rsqrt(jnp.maximum(d[:, 0:1] + d[:, 2:3], 1.0))
        si = lax.rsqrt(jnp.maximum(d[:, 1:2] + d[:, 3:4], 1.0))
        xw = jnp.dot(x_ref[...], w_ref[...],
                     preferred_element_type=jnp.float32)
        hw_ref[...] = xw * so
        s_ref[...] = jnp.concatenate([so, si], axis=1)

    return pl.pallas_call(
        body,
        grid=(NP // RB,),
        in_specs=[
            pl.BlockSpec((RB, D_IN), lambda i: (i, 0)),
            pl.BlockSpec((D_IN, D_HID), lambda i: (0, 0)),
            pl.BlockSpec((RB, 4), lambda i: (i, 0)),
        ],
        out_specs=[
            pl.BlockSpec((RB, D_HID), lambda i: (i, 0)),
            pl.BlockSpec((RB, 2), lambda i: (i, 0)),
        ],
        out_shape=[
            jax.ShapeDtypeStruct((NP, D_HID), jnp.float32),
            jax.ShapeDtypeStruct((NP, 2), jnp.float32),
        ],
    )(xp, W1, deg4)


def _tc_mid(a0, a1, S, b1, W2):
    def body(a0_ref, a1_ref, s_ref, b_ref, w_ref, o_ref):
        sv = s_ref[...]
        a = a0_ref[...] + a1_ref[...]
        h = jnp.maximum(a * sv[:, 1:2] + b_ref[...], 0.0)
        o_ref[...] = jnp.dot(h, w_ref[...],
                             preferred_element_type=jnp.float32) * sv[:, 0:1]

    return pl.pallas_call(
        body,
        grid=(NP // RB,),
        in_specs=[
            pl.BlockSpec((RB, D_HID), lambda i: (i, 0)),
            pl.BlockSpec((RB, D_HID), lambda i: (i, 0)),
            pl.BlockSpec((RB, 2), lambda i: (i, 0)),
            pl.BlockSpec((1, D_HID), lambda i: (0, 0)),
            pl.BlockSpec((D_HID, D_HID), lambda i: (0, 0)),
        ],
        out_specs=pl.BlockSpec((RB, D_HID), lambda i: (i, 0)),
        out_shape=jax.ShapeDtypeStruct((NP, D_HID), jnp.float32),
    )(a0, a1, S, b1, W2)


def _tc_post(a0, a1, S, b2):
    def body(a0_ref, a1_ref, s_ref, b_ref, o_ref):
        sv = s_ref[...]
        a = a0_ref[...] + a1_ref[...]
        o_ref[...] = jnp.maximum(a * sv[:, 1:2] + b_ref[...], 0.0)

    return pl.pallas_call(
        body,
        grid=(NP // RB,),
        in_specs=[
            pl.BlockSpec((RB, D_HID), lambda i: (i, 0)),
            pl.BlockSpec((RB, D_HID), lambda i: (i, 0)),
            pl.BlockSpec((RB, 2), lambda i: (i, 0)),
            pl.BlockSpec((1, D_HID), lambda i: (0, 0)),
        ],
        out_specs=pl.BlockSpec((RB, D_HID), lambda i: (i, 0)),
        out_shape=jax.ShapeDtypeStruct((NP, D_HID), jnp.float32),
    )(a0, a1, S, b2)


def _tc_head(xrp, WdP, bdP):
    def body(x_ref, w_ref, b_ref, o_ref):
        o_ref[...] = jnp.dot(x_ref[...], w_ref[...],
                             preferred_element_type=jnp.float32) + b_ref[...]

    return pl.pallas_call(
        body,
        in_specs=[
            pl.BlockSpec((2560, 4 * D_HID), lambda: (0, 0)),
            pl.BlockSpec((4 * D_HID, 8), lambda: (0, 0)),
            pl.BlockSpec((1, 8), lambda: (0, 0)),
        ],
        out_specs=pl.BlockSpec((2560, 8), lambda: (0, 0)),
        out_shape=jax.ShapeDtypeStruct((2560, 8), jnp.float32),
    )(xrp, WdP, bdP)


def kernel(x, edge_index, W1, b1, W2, b2, Wd, bd):
    f32 = jnp.float32
    src = edge_index[0].astype(jnp.int32)
    dst = edge_index[1].astype(jnp.int32)
    pad = EP - E
    src_t = jnp.concatenate([src, jnp.full((pad,), N, jnp.int32)]
                            ).reshape(NW, KJ, CHUNK)
    dst_t = jnp.concatenate([dst, jnp.full((pad,), N, jnp.int32)]
                            ).reshape(NW, KJ, CHUNK)
    ones128 = jnp.ones((CHUNK,), f32)
    zerosN = jnp.zeros((NP,), f32)
    zrows = jnp.zeros((NP, D_HID), f32)

    deg = _make_degree_kernel()(src_t, dst_t, ones128, zerosN)
    deg4 = deg.transpose(2, 0, 1).reshape(NP, 2 * NC)

    xp = jnp.pad(x, ((0, NP - N), (0, 0)))
    b1r = b1.reshape(1, D_HID)
    b2r = b2.reshape(1, D_HID)

    hw1, S = _tc_layer1(xp, W1, deg4)
    conv = _make_conv_kernel()
    agg1 = conv(hw1, src_t, dst_t, zrows)
    hw2 = _tc_mid(agg1[0], agg1[1], S, b1r, W2)
    agg2 = conv(hw2, src_t, dst_t, zrows)
    h2 = _tc_post(agg2[0], agg2[1], S, b2r)

    xr = h2[:N].reshape(N // 4, 4 * D_HID)
    xrp = jnp.pad(xr, ((0, 2560 - N // 4), (0, 0)))
    WdP = jnp.pad(Wd, ((0, 0), (0, 7)))
    bdP = jnp.pad(bd, (0, 7)).reshape(1, 8)
    out = _tc_head(xrp, WdP, bdP)
    return out[:N // 4, :1]


# speedup vs baseline: 1.1481x; 1.0212x over previous
import functools

import jax
import jax.numpy as jnp
from jax import lax
from jax.experimental import pallas as pl
from jax.experimental.pallas import tpu as pltpu
from jax.experimental.pallas import tpu_sc as plsc

N = 10000
NP = 10240
E = 320000
D_IN = 128
D_HID = 24
NC, NS = 2, 16
NW = NC * NS
CHUNK = 128
KJ = 80
EP = NW * KJ * CHUNK
RPT = NP // NS


def _mesh():
    return plsc.VectorSubcoreMesh(core_axis_name="c", subcore_axis_name="s",
                                  num_cores=NC, num_subcores=NS)


@functools.cache
def _make_degree_kernel():
    @functools.partial(
        pl.kernel,
        out_type=jax.ShapeDtypeStruct((NC, 2, NP), jnp.float32),
        mesh=_mesh(),
        scratch_types=[
            pltpu.VMEM((KJ, CHUNK), jnp.int32),
            pltpu.VMEM((KJ, CHUNK), jnp.int32),
            pltpu.VMEM((CHUNK,), jnp.float32),
            pltpu.VMEM_SHARED((NP,), jnp.float32),
            pltpu.VMEM_SHARED((NP,), jnp.float32),
            pltpu.SemaphoreType.DMA,
            pltpu.SemaphoreType.DMA,
        ],
        compiler_params=pltpu.CompilerParams(use_tc_tiling_on_sc=False),
    )
    def _degree_kernel(srcw, dstw, ones_hbm, zeros_hbm, out, src_v, dst_v,
                       ones_v, dego, degi, sem0, sem1):
        c = lax.axis_index("c")
        s = lax.axis_index("s")
        w = c * NS + s
        base = s * RPT
        pltpu.sync_copy(zeros_hbm.at[pl.ds(base, RPT)],
                        dego.at[pl.ds(base, RPT)])
        pltpu.sync_copy(zeros_hbm.at[pl.ds(base, RPT)],
                        degi.at[pl.ds(base, RPT)])
        pltpu.sync_copy(ones_hbm, ones_v)
        pltpu.sync_copy(srcw.at[w], src_v)
        pltpu.sync_copy(dstw.at[w], dst_v)
        plsc.subcore_barrier()

        @pl.loop(0, KJ)
        def _(j):
            a = pltpu.async_copy(ones_v, dego.at[src_v.at[j]], sem0,
                                 add=True)
            b = pltpu.async_copy(ones_v, degi.at[dst_v.at[j]], sem1,
                                 add=True)
            a.wait()
            b.wait()

        plsc.subcore_barrier()
        pltpu.sync_copy(dego.at[pl.ds(base, RPT)],
                        out.at[c, 0, pl.ds(base, RPT)])
        pltpu.sync_copy(degi.at[pl.ds(base, RPT)],
                        out.at[c, 1, pl.ds(base, RPT)])

    return _degree_kernel


@functools.cache
def _make_conv_kernel():
    @functools.partial(
        pl.kernel,
        out_type=jax.ShapeDtypeStruct((NC, NP, D_HID), jnp.float32),
        mesh=_mesh(),
        scratch_types=[
            pltpu.VMEM((KJ, CHUNK), jnp.int32),
            pltpu.VMEM((KJ, CHUNK), jnp.int32),
            pltpu.VMEM((CHUNK, D_HID), jnp.float32),
            pltpu.VMEM((CHUNK, D_HID), jnp.float32),
            pltpu.VMEM_SHARED((NP, D_HID), jnp.float32),
            pltpu.VMEM_SHARED((NP, D_HID), jnp.float32),
            pltpu.SemaphoreType.DMA,
            pltpu.SemaphoreType.DMA,
        ],
        compiler_params=pltpu.CompilerParams(use_tc_tiling_on_sc=False),
    )
    def _conv_kernel(hw, srcw, dstw, zrows, out, src_v, dst_v, rows0, rows1,
                     accum, hw_sp, sem0, sem1):
        c = lax.axis_index("c")
        s = lax.axis_index("s")
        w = c * NS + s
        base = s * RPT
        pltpu.sync_copy(zrows.at[pl.ds(base, RPT)],
                        accum.at[pl.ds(base, RPT)])
        pltpu.sync_copy(hw.at[pl.ds(base, RPT)], hw_sp.at[pl.ds(base, RPT)])
        pltpu.sync_copy(srcw.at[w], src_v)
        pltpu.sync_copy(dstw.at[w], dst_v)
        plsc.subcore_barrier()

        pltpu.async_copy(hw_sp.at[src_v.at[0]], rows0, sem0)

        @pl.loop(0, KJ // 2)
        def _(p):
            j = 2 * p
            pltpu.make_async_copy(hw_sp.at[src_v.at[j]], rows0, sem0).wait()
            pltpu.async_copy(hw_sp.at[src_v.at[j + 1]], rows1, sem1)
            pltpu.sync_copy(rows0, accum.at[dst_v.at[j]], add=True)

            @pl.when(j + 2 < KJ)
            def _():
                pltpu.async_copy(hw_sp.at[src_v.at[j + 2]], rows0, sem0)

            pltpu.make_async_copy(hw_sp.at[src_v.at[j + 1]], rows1,
                                  sem1).wait()
            pltpu.sync_copy(rows1, accum.at[dst_v.at[j + 1]], add=True)

        plsc.subcore_barrier()
        pltpu.sync_copy(accum.at[pl.ds(base, RPT)],
                        out.at[c, pl.ds(base, RPT)])

    return _conv_kernel


RB = 2560


def _tc_layer1(xp, W1, deg4):
    def body(x_ref, w_ref, d_ref, hw_ref, s_ref):
        d = d_ref[...]
        so = lax.rsqrt(jnp.maximum(d[:, 0:1] + d[:, 2:3], 1.0))
        si = lax.rsqrt(jnp.maximum(d[:, 1:2] + d[:, 3:4], 1.0))
        xw = jnp.dot(x_ref[...], w_ref[...],
                     preferred_element_type=jnp.float32)
        hw_ref[...] = xw * so
        s_ref[...] = jnp.concatenate([so, si], axis=1)

    return pl.pallas_call(
        body,
        grid=(NP // RB,),
        in_specs=[
            pl.BlockSpec((RB, D_IN), lambda i: (i, 0)),
            pl.BlockSpec((D_IN, D_HID), lambda i: (0, 0)),
            pl.BlockSpec((RB, 4), lambda i: (i, 0)),
        ],
        out_specs=[
            pl.BlockSpec((RB, D_HID), lambda i: (i, 0)),
            pl.BlockSpec((RB, 2), lambda i: (i, 0)),
        ],
        out_shape=[
            jax.ShapeDtypeStruct((NP, D_HID), jnp.float32),
            jax.ShapeDtypeStruct((NP, 2), jnp.float32),
        ],
    )(xp, W1, deg4)


def _tc_mid(a0, a1, S, b1, W2):
    def body(a0_ref, a1_ref, s_ref, b_ref, w_ref, o_ref):
        sv = s_ref[...]
        a = a0_ref[...] + a1_ref[...]
        h = jnp.maximum(a * sv[:, 1:2] + b_ref[...], 0.0)
        o_ref[...] = jnp.dot(h, w_ref[...],
                             preferred_element_type=jnp.float32) * sv[:, 0:1]

    return pl.pallas_call(
        body,
        grid=(NP // RB,),
        in_specs=[
            pl.BlockSpec((RB, D_HID), lambda i: (i, 0)),
            pl.BlockSpec((RB, D_HID), lambda i: (i, 0)),
            pl.BlockSpec((RB, 2), lambda i: (i, 0)),
            pl.BlockSpec((1, D_HID), lambda i: (0, 0)),
            pl.BlockSpec((D_HID, D_HID), lambda i: (0, 0)),
        ],
        out_specs=pl.BlockSpec((RB, D_HID), lambda i: (i, 0)),
        out_shape=jax.ShapeDtypeStruct((NP, D_HID), jnp.float32),
    )(a0, a1, S, b1, W2)


def _tc_post(a0, a1, S, b2):
    def body(a0_ref, a1_ref, s_ref, b_ref, o_ref):
        sv = s_ref[...]
        a = a0_ref[...] + a1_ref[...]
        o_ref[...] = jnp.maximum(a * sv[:, 1:2] + b_ref[...], 0.0)

    return pl.pallas_call(
        body,
        grid=(NP // RB,),
        in_specs=[
            pl.BlockSpec((RB, D_HID), lambda i: (i, 0)),
            pl.BlockSpec((RB, D_HID), lambda i: (i, 0)),
            pl.BlockSpec((RB, 2), lambda i: (i, 0)),
            pl.BlockSpec((1, D_HID), lambda i: (0, 0)),
        ],
        out_specs=pl.BlockSpec((RB, D_HID), lambda i: (i, 0)),
        out_shape=jax.ShapeDtypeStruct((NP, D_HID), jnp.float32),
    )(a0, a1, S, b2)


def _tc_head(xrp, WdP, bdP):
    def body(x_ref, w_ref, b_ref, o_ref):
        o_ref[...] = jnp.dot(x_ref[...], w_ref[...],
                             preferred_element_type=jnp.float32) + b_ref[...]

    return pl.pallas_call(
        body,
        in_specs=[
            pl.BlockSpec((2560, 4 * D_HID), lambda: (0, 0)),
            pl.BlockSpec((4 * D_HID, 8), lambda: (0, 0)),
            pl.BlockSpec((1, 8), lambda: (0, 0)),
        ],
        out_specs=pl.BlockSpec((2560, 8), lambda: (0, 0)),
        out_shape=jax.ShapeDtypeStruct((2560, 8), jnp.float32),
    )(xrp, WdP, bdP)


def kernel(x, edge_index, W1, b1, W2, b2, Wd, bd):
    f32 = jnp.float32
    src = edge_index[0].astype(jnp.int32)
    dst = edge_index[1].astype(jnp.int32)
    pad = EP - E
    src_t = jnp.concatenate([src, jnp.full((pad,), N, jnp.int32)]
                            ).reshape(NW, KJ, CHUNK)
    dst_t = jnp.concatenate([dst, jnp.full((pad,), N, jnp.int32)]
                            ).reshape(NW, KJ, CHUNK)
    ones128 = jnp.ones((CHUNK,), f32)
    zerosN = jnp.zeros((NP,), f32)
    zrows = jnp.zeros((NP, D_HID), f32)

    deg = _make_degree_kernel()(src_t, dst_t, ones128, zerosN)
    deg4 = deg.transpose(2, 0, 1).reshape(NP, 2 * NC)

    xp = jnp.pad(x, ((0, NP - N), (0, 0)))
    b1r = b1.reshape(1, D_HID)
    b2r = b2.reshape(1, D_HID)

    hw1, S = _tc_layer1(xp, W1, deg4)
    conv = _make_conv_kernel()
    agg1 = conv(hw1, src_t, dst_t, zrows)
    hw2 = _tc_mid(agg1[0], agg1[1], S, b1r, W2)
    agg2 = conv(hw2, src_t, dst_t, zrows)
    h2 = _tc_post(agg2[0], agg2[1], S, b2r)

    xr = h2[:N].reshape(N // 4, 4 * D_HID)
    xrp = jnp.pad(xr, ((0, 2560 - N // 4), (0, 0)))
    WdP = jnp.pad(Wd, ((0, 0), (0, 7)))
    bdP = jnp.pad(bd, (0, 7)).reshape(1, 8)
    out = _tc_head(xrp, WdP, bdP)
    return out[:N // 4, :1]
